# Initial kernel scaffold; baseline (speedup 1.0000x reference)
import jax
import jax.numpy as jnp
from jax.experimental import pallas as pl


def kernel(nodes_embed, relation_embed, edges, w_neighbor):
    raise NotImplementedError("write your pallas kernel here")


# speedup vs baseline: 4.1460x; 4.1460x over previous
import functools

import jax
import jax.numpy as jnp
from jax import lax
from jax.experimental import pallas as pl
from jax.experimental.pallas import tpu as pltpu
from jax.experimental.pallas import tpu_sc as plsc

NC = 2
NS = 16
NW = NC * NS

CHUNK = 80
BR = 32


def _sc_aggregate(n_nodes, d, n_edges):
  assert n_edges % (NW * CHUNK) == 0
  edges_per_worker = n_edges // NW
  n_chunks = edges_per_worker // CHUNK
  n_pad = ((n_nodes + NS * BR - 1) // (NS * BR)) * (NS * BR)
  rows_per_tile = n_pad // NS
  assert rows_per_tile % 128 == 0

  mesh = plsc.VectorSubcoreMesh(core_axis_name="c", subcore_axis_name="s")

  @functools.partial(
      pl.kernel,
      out_type=(
          jax.ShapeDtypeStruct((NC, n_pad, d), jnp.float32),
          jax.ShapeDtypeStruct((NC * n_pad,), jnp.float32),
      ),
      mesh=mesh,
      scratch_types=[
          pltpu.VMEM_SHARED((n_pad, d), jnp.float32),
          pltpu.VMEM_SHARED((n_pad,), jnp.float32),
          pltpu.VMEM((CHUNK,), jnp.int32),
          pltpu.VMEM((CHUNK,), jnp.int32),
          pltpu.VMEM((CHUNK,), jnp.int32),
          pltpu.VMEM((CHUNK, d), jnp.float32),
          pltpu.VMEM((CHUNK, d), jnp.float32),
          pltpu.VMEM((CHUNK,), jnp.float32),
          pltpu.VMEM((BR, d), jnp.float32),
          pltpu.VMEM((rows_per_tile,), jnp.float32),
          pltpu.SemaphoreType.DMA,
          pltpu.SemaphoreType.DMA,
      ],
  )
  def agg_kernel(nodes_hbm, rel_hbm, srci_hbm, reli_hbm, dsti_hbm,
                 zrow_hbm, zcnt_hbm, ones_hbm,
                 sums_out, counts_out,
                 sums_sp, cnts_sp, src_v, rel_v, dst_v, bufx, bufr, ones_v,
                 bounce, bounce_c, sem0, sem1):
    c = lax.axis_index("c")
    s = lax.axis_index("s")
    wid = c * NS + s
    base = wid * edges_per_worker
    row0 = s * rows_per_tile

    pltpu.sync_copy(zrow_hbm, bounce)
    pltpu.sync_copy(zcnt_hbm, bounce_c)
    pltpu.sync_copy(ones_hbm, ones_v)

    def zinit_body(j, carry):
      r = row0 + j * BR
      pltpu.sync_copy(bounce, sums_sp.at[pl.ds(r, BR)])
      return carry

    lax.fori_loop(0, rows_per_tile // BR, zinit_body, 0)
    pltpu.sync_copy(bounce_c, cnts_sp.at[pl.ds(row0, rows_per_tile)])
    plsc.subcore_barrier()

    def chunk_body(i, carry):
      off = base + i * CHUNK
      pltpu.sync_copy(srci_hbm.at[pl.ds(off, CHUNK)], src_v)
      pltpu.sync_copy(reli_hbm.at[pl.ds(off, CHUNK)], rel_v)
      pltpu.sync_copy(dsti_hbm.at[pl.ds(off, CHUNK)], dst_v)
      gx = pltpu.async_copy(nodes_hbm.at[src_v], bufx, sem0)
      gr = pltpu.async_copy(rel_hbm.at[rel_v], bufr, sem1)
      gx.wait()
      gr.wait()
      pltpu.sync_copy(bufx, sums_sp.at[dst_v], add=True)
      pltpu.sync_copy(bufr, sums_sp.at[dst_v], add=True)
      pltpu.sync_copy(ones_v, cnts_sp.at[dst_v], add=True)
      return carry

    lax.fori_loop(0, n_chunks, chunk_body, 0)
    plsc.subcore_barrier()

    def writeback_body(j, carry):
      r = row0 + j * BR
      pltpu.sync_copy(sums_sp.at[pl.ds(r, BR)], bounce)
      pltpu.sync_copy(bounce, sums_out.at[c, pl.ds(r, BR)])
      return carry

    lax.fori_loop(0, rows_per_tile // BR, writeback_body, 0)
    pltpu.sync_copy(cnts_sp.at[pl.ds(row0, rows_per_tile)], bounce_c)
    pltpu.sync_copy(bounce_c,
                    counts_out.at[pl.ds(c * n_pad + row0, rows_per_tile)])

  return agg_kernel


def _combine_body(nodes_ref, sums_ref, counts_ref, w_ref, out_ref):
  s = sums_ref[0] + sums_ref[1]
  cnt = counts_ref[0, :, 0:1] + counts_ref[1, :, 0:1]
  cnt = jnp.maximum(cnt, 1.0)
  agg = jnp.dot(s, w_ref[...], preferred_element_type=jnp.float32)
  out_ref[...] = nodes_ref[...] + agg / cnt


def kernel(nodes_embed, relation_embed, edges, w_neighbor):
  n_nodes, d = nodes_embed.shape
  n_edges = edges.shape[0]

  src = edges[:, 0]
  rel = edges[:, 1]
  dst = edges[:, 2]

  n_pad = ((n_nodes + NS * BR - 1) // (NS * BR)) * (NS * BR)
  rows_per_tile = n_pad // NS
  zrow = jnp.zeros((BR, d), jnp.float32)
  zcnt = jnp.zeros((rows_per_tile,), jnp.float32)
  ones = jnp.ones((CHUNK,), jnp.float32)

  agg_kernel = _sc_aggregate(n_nodes, d, n_edges)
  sums, counts = agg_kernel(nodes_embed, relation_embed, src, rel, dst,
                            zrow, zcnt, ones)
  counts = counts.reshape(NC, n_pad)[..., None]

  block_rows = 400
  n_blocks = n_nodes // block_rows
  h = pl.pallas_call(
      _combine_body,
      grid=(n_blocks,),
      in_specs=[
          pl.BlockSpec((block_rows, d), lambda i: (i, 0)),
          pl.BlockSpec((NC, block_rows, d), lambda i: (0, i, 0)),
          pl.BlockSpec((NC, block_rows, 1), lambda i: (0, i, 0)),
          pl.BlockSpec((d, d), lambda i: (0, 0)),
      ],
      out_specs=pl.BlockSpec((block_rows, d), lambda i: (i, 0)),
      out_shape=jax.ShapeDtypeStruct((n_nodes, d), jnp.float32),
  )(nodes_embed, sums, counts, w_neighbor)
  return h


# speedup vs baseline: 6.0142x; 1.4506x over previous
import functools

import jax
import jax.numpy as jnp
from jax import lax
from jax.experimental import pallas as pl
from jax.experimental.pallas import tpu as pltpu
from jax.experimental.pallas import tpu_sc as plsc

NC = 2
NS = 16
NW = NC * NS

CHUNK = 80
BR = 32


def _sc_aggregate(n_nodes, d, n_edges):
  assert n_edges % (NW * CHUNK) == 0
  edges_per_worker = n_edges // NW
  n_chunks = edges_per_worker // CHUNK
  n_pad = ((n_nodes + NS * BR - 1) // (NS * BR)) * (NS * BR)
  rows_per_tile = n_pad // NS
  assert rows_per_tile % 128 == 0

  mesh = plsc.VectorSubcoreMesh(core_axis_name="c", subcore_axis_name="s")

  @functools.partial(
      pl.kernel,
      out_type=(
          jax.ShapeDtypeStruct((NC, n_pad, d), jnp.float32),
          jax.ShapeDtypeStruct((NC * n_pad,), jnp.float32),
      ),
      mesh=mesh,
      scratch_types=[
          pltpu.VMEM_SHARED((n_pad, d), jnp.float32),
          pltpu.VMEM_SHARED((n_pad,), jnp.float32),
          pltpu.VMEM((CHUNK,), jnp.int32),
          pltpu.VMEM((CHUNK,), jnp.int32),
          pltpu.VMEM((CHUNK,), jnp.int32),
          pltpu.VMEM((CHUNK, d), jnp.float32),
          pltpu.VMEM((CHUNK, d), jnp.float32),
          pltpu.VMEM((CHUNK,), jnp.int32),
          pltpu.VMEM((CHUNK,), jnp.int32),
          pltpu.VMEM((CHUNK,), jnp.int32),
          pltpu.VMEM((CHUNK, d), jnp.float32),
          pltpu.VMEM((CHUNK, d), jnp.float32),
          pltpu.VMEM((CHUNK,), jnp.float32),
          pltpu.VMEM((BR, d), jnp.float32),
          pltpu.VMEM((rows_per_tile,), jnp.float32),
          pltpu.SemaphoreType.DMA,
          pltpu.SemaphoreType.DMA,
          pltpu.SemaphoreType.DMA,
          pltpu.SemaphoreType.DMA,
      ],
  )
  def agg_kernel(nodes_hbm, rel_hbm, srci_hbm, reli_hbm, dsti_hbm,
                 zrow_hbm, zcnt_hbm, ones_hbm,
                 sums_out, counts_out,
                 sums_sp, cnts_sp,
                 src_a, rel_a, dst_a, bufx_a, bufr_a,
                 src_b, rel_b, dst_b, bufx_b, bufr_b,
                 ones_v, bounce, bounce_c,
                 semx_a, semr_a, semx_b, semr_b):
    c = lax.axis_index("c")
    s = lax.axis_index("s")
    wid = c * NS + s
    base = wid * edges_per_worker
    row0 = s * rows_per_tile

    buf_a = (src_a, rel_a, dst_a, bufx_a, bufr_a, semx_a, semr_a)
    buf_b = (src_b, rel_b, dst_b, bufx_b, bufr_b, semx_b, semr_b)

    def start(buf, i):
      src_v, rel_v, dst_v, bufx, bufr, semx, semr = buf
      off = base + i * CHUNK
      pltpu.sync_copy(srci_hbm.at[pl.ds(off, CHUNK)], src_v)
      pltpu.sync_copy(reli_hbm.at[pl.ds(off, CHUNK)], rel_v)
      pltpu.sync_copy(dsti_hbm.at[pl.ds(off, CHUNK)], dst_v)
      pltpu.async_copy(nodes_hbm.at[src_v], bufx, semx)
      pltpu.async_copy(rel_hbm.at[rel_v], bufr, semr)

    def finish(buf):
      src_v, rel_v, dst_v, bufx, bufr, semx, semr = buf
      pltpu.make_async_copy(nodes_hbm.at[src_v], bufx, semx).wait()
      pltpu.make_async_copy(rel_hbm.at[rel_v], bufr, semr).wait()
      pltpu.sync_copy(bufx, sums_sp.at[dst_v], add=True)
      pltpu.sync_copy(bufr, sums_sp.at[dst_v], add=True)
      pltpu.sync_copy(ones_v, cnts_sp.at[dst_v], add=True)

    pltpu.sync_copy(zrow_hbm, bounce)
    pltpu.sync_copy(zcnt_hbm, bounce_c)
    pltpu.sync_copy(ones_hbm, ones_v)

    def zinit_body(j, carry):
      r = row0 + j * BR
      pltpu.sync_copy(bounce, sums_sp.at[pl.ds(r, BR)])
      return carry

    lax.fori_loop(0, rows_per_tile // BR, zinit_body, 0)
    pltpu.sync_copy(bounce_c, cnts_sp.at[pl.ds(row0, rows_per_tile)])
    plsc.subcore_barrier()

    assert n_chunks >= 4 and n_chunks % 2 == 1
    n_pairs = (n_chunks - 3) // 2

    start(buf_a, 0)
    start(buf_b, 1)

    def pair_body(j, carry):
      finish(buf_a)
      start(buf_a, 2 * j + 2)
      finish(buf_b)
      start(buf_b, 2 * j + 3)
      return carry

    lax.fori_loop(0, n_pairs, pair_body, 0)
    finish(buf_a)
    start(buf_a, n_chunks - 1)
    finish(buf_b)
    finish(buf_a)
    plsc.subcore_barrier()

    def writeback_body(j, carry):
      r = row0 + j * BR
      pltpu.sync_copy(sums_sp.at[pl.ds(r, BR)], bounce)
      pltpu.sync_copy(bounce, sums_out.at[c, pl.ds(r, BR)])
      return carry

    lax.fori_loop(0, rows_per_tile // BR, writeback_body, 0)
    pltpu.sync_copy(cnts_sp.at[pl.ds(row0, rows_per_tile)], bounce_c)
    pltpu.sync_copy(bounce_c,
                    counts_out.at[pl.ds(c * n_pad + row0, rows_per_tile)])

  return agg_kernel


def _combine_body(nodes_ref, sums_ref, counts_ref, w_ref, out_ref):
  s = sums_ref[0] + sums_ref[1]
  cnt = counts_ref[0, :, 0:1] + counts_ref[1, :, 0:1]
  cnt = jnp.maximum(cnt, 1.0)
  agg = jnp.dot(s, w_ref[...], preferred_element_type=jnp.float32)
  out_ref[...] = nodes_ref[...] + agg / cnt


def kernel(nodes_embed, relation_embed, edges, w_neighbor):
  n_nodes, d = nodes_embed.shape
  n_edges = edges.shape[0]

  src = edges[:, 0]
  rel = edges[:, 1]
  dst = edges[:, 2]

  n_pad = ((n_nodes + NS * BR - 1) // (NS * BR)) * (NS * BR)
  rows_per_tile = n_pad // NS
  zrow = jnp.zeros((BR, d), jnp.float32)
  zcnt = jnp.zeros((rows_per_tile,), jnp.float32)
  ones = jnp.ones((CHUNK,), jnp.float32)

  agg_kernel = _sc_aggregate(n_nodes, d, n_edges)
  sums, counts = agg_kernel(nodes_embed, relation_embed, src, rel, dst,
                            zrow, zcnt, ones)
  counts = counts.reshape(NC, n_pad)[..., None]

  block_rows = 400
  n_blocks = n_nodes // block_rows
  h = pl.pallas_call(
      _combine_body,
      grid=(n_blocks,),
      in_specs=[
          pl.BlockSpec((block_rows, d), lambda i: (i, 0)),
          pl.BlockSpec((NC, block_rows, d), lambda i: (0, i, 0)),
          pl.BlockSpec((NC, block_rows, 1), lambda i: (0, i, 0)),
          pl.BlockSpec((d, d), lambda i: (0, 0)),
      ],
      out_specs=pl.BlockSpec((block_rows, d), lambda i: (i, 0)),
      out_shape=jax.ShapeDtypeStruct((n_nodes, d), jnp.float32),
  )(nodes_embed, sums, counts, w_neighbor)
  return h
